# baseline (speedup 1.0000x reference)
import functools

import jax
import jax.numpy as jnp
from jax import lax
from jax.experimental import pallas as pl
from jax.experimental.pallas import tpu as pltpu
from jax.experimental.pallas import tpu_sc as plsc

DIM = 64
B0, B1 = 16384, 26
B_TOTAL = B0 * B1
NUM_WORKERS = 32
PER_W = B_TOTAL // NUM_WORKERS
CHUNK = 128
N_CHUNKS = PER_W // CHUNK


def _gather_body(table_hbm, idx_hbm, out_hbm, idx_v, rows_v, sem):
    wid = lax.axis_index("s") * 2 + lax.axis_index("c")
    base = pl.multiple_of(wid * PER_W, PER_W)
    pltpu.sync_copy(idx_hbm.at[pl.ds(base, PER_W)], idx_v)

    def body(i, carry):
        off = pl.multiple_of(i * CHUNK, CHUNK)
        pltpu.async_copy(
            table_hbm.at[idx_v.at[pl.ds(off, CHUNK)]], rows_v, sem
        ).wait()
        pltpu.sync_copy(rows_v, out_hbm.at[pl.ds(base + off, CHUNK)])
        return carry

    lax.fori_loop(0, N_CHUNKS, body, 0)


def kernel(X, embeddings):
    idx = X.reshape(-1)
    mesh = plsc.VectorSubcoreMesh(core_axis_name="c", subcore_axis_name="s")
    out = pl.kernel(
        _gather_body,
        out_type=jax.ShapeDtypeStruct((B_TOTAL, DIM), jnp.float32),
        mesh=mesh,
        scratch_types=[
            pltpu.VMEM((PER_W,), jnp.int32),
            pltpu.VMEM((CHUNK, DIM), jnp.float32),
            pltpu.SemaphoreType.DMA,
        ],
        compiler_params=pltpu.CompilerParams(use_tc_tiling_on_sc=False),
    )(embeddings, idx)
    return out.reshape(B0, B1, DIM)


# speedup vs baseline: 1.0775x; 1.0775x over previous
import functools

import jax
import jax.numpy as jnp
from jax import lax
from jax.experimental import pallas as pl
from jax.experimental.pallas import tpu as pltpu
from jax.experimental.pallas import tpu_sc as plsc

DIM = 64
B0, B1 = 16384, 26
B_TOTAL = B0 * B1
NUM_WORKERS = 32
PER_W = B_TOTAL // NUM_WORKERS
CHUNK = 512
N_CHUNKS = PER_W // CHUNK
N_PAIRS = N_CHUNKS // 2


def _gather_body(table_hbm, idx_hbm, out_hbm, idx_v, rows0, rows1, sem0, sem1):
    wid = lax.axis_index("s") * 2 + lax.axis_index("c")
    base = pl.multiple_of(wid * PER_W, PER_W)
    pltpu.sync_copy(idx_hbm.at[pl.ds(base, PER_W)], idx_v)

    def start_gather(i, rows, sem):
        off = pl.multiple_of(i * CHUNK, CHUNK)
        pltpu.async_copy(table_hbm.at[idx_v.at[pl.ds(off, CHUNK)]], rows, sem)

    def wait_gather(rows, sem):
        pltpu.make_async_copy(table_hbm.at[pl.ds(0, CHUNK)], rows, sem).wait()

    def write(i, rows):
        off = pl.multiple_of(i * CHUNK, CHUNK)
        pltpu.sync_copy(rows, out_hbm.at[pl.ds(base + off, CHUNK)])

    start_gather(0, rows0, sem0)

    def pair_body(p, carry):
        i0 = p * 2
        start_gather(i0 + 1, rows1, sem1)
        wait_gather(rows0, sem0)
        write(i0, rows0)

        @pl.when(p < N_PAIRS - 1)
        def _():
            start_gather(i0 + 2, rows0, sem0)

        wait_gather(rows1, sem1)
        write(i0 + 1, rows1)
        return carry

    lax.fori_loop(0, N_PAIRS, pair_body, 0)


def kernel(X, embeddings):
    idx = X.reshape(-1)
    mesh = plsc.VectorSubcoreMesh(core_axis_name="c", subcore_axis_name="s")
    out = pl.kernel(
        _gather_body,
        out_type=jax.ShapeDtypeStruct((B_TOTAL, DIM), jnp.float32),
        mesh=mesh,
        scratch_types=[
            pltpu.VMEM((PER_W,), jnp.int32),
            pltpu.VMEM((CHUNK, DIM), jnp.float32),
            pltpu.VMEM((CHUNK, DIM), jnp.float32),
            pltpu.SemaphoreType.DMA,
            pltpu.SemaphoreType.DMA,
        ],
        compiler_params=pltpu.CompilerParams(use_tc_tiling_on_sc=False),
    )(embeddings, idx)
    return out.reshape(B0, B1, DIM)
